# baseline (speedup 1.0000x reference)
import functools
import math

import jax
import jax.numpy as jnp
from jax import lax
from jax.experimental import pallas as pl
from jax.experimental.pallas import tpu as pltpu
from jax.experimental.pallas import tpu_sc as plsc

N = 10000
F = 512
K = 5000
NW = 32
NG = K // 8
IDXPAD = 5120
MININT = -(2**31)


def _one_out(x):
    return x[0] if isinstance(x, (list, tuple)) else x


def _score_gate_body(x_ref, p_ref, xg_ref, y_ref):
    x = x_ref[...]
    p = p_ref[...]
    kn = p / jnp.sqrt(jnp.sum(p * p))
    y = jnp.dot(x, kn, preferred_element_type=jnp.float32)
    xg_ref[...] = x * jnp.tanh(y)
    y_ref[...] = y


def _score_gate(x, p):
    return pl.pallas_call(
        _score_gate_body,
        out_shape=[
            jax.ShapeDtypeStruct((N, F), jnp.float32),
            jax.ShapeDtypeStruct((N, 1), jnp.float32),
        ],
    )(x, p)


def _threshold_body(y_ref, meta_ref):
    yv = y_ref[...]
    s = lax.bitcast_convert_type(yv, jnp.int32)
    key = jnp.where(s < 0, s ^ jnp.int32(0x7FFFFFFF), s)

    def bit_body(b, cand):
        bit = 31 - b
        c2 = cand | (jnp.int32(1) << bit)
        cnt = jnp.sum((key >= (c2 ^ jnp.int32(MININT))).astype(jnp.int32))
        return lax.select(cnt >= K, c2, cand)

    cand = lax.fori_loop(0, 32, bit_body, jnp.int32(0))
    ts = cand ^ jnp.int32(MININT)
    cnt_gt = jnp.sum((key > ts).astype(jnp.int32))
    budget = jnp.int32(K) - cnt_gt
    rows = lax.broadcasted_iota(jnp.int32, (8, 128), 0)
    meta_ref[...] = jnp.where(rows == 0, ts, jnp.where(rows == 1, budget, jnp.int32(0)))


def _threshold(y2):
    return pl.pallas_call(
        _threshold_body,
        out_shape=jax.ShapeDtypeStruct((8, 128), jnp.int32),
    )(y2)


def _sc_compact_body(sm_hbm, idx_hbm, sv, idx_v, idx2_v):
    if True:
        pltpu.sync_copy(sm_hbm, sv)
        thr = sv[pl.ds(N, 16)]
        bud = sv[pl.ds(N + 16, 16)]
        iota = lax.iota(jnp.int32, 16)
        one = jnp.broadcast_to(jnp.int32(1), (16,))
        zero = jnp.broadcast_to(jnp.int32(0), (16,))

        def init_body(j, carry):
            idx_v[pl.ds(j * 16, 16)] = j * 16 + iota
            return carry

        lax.fori_loop(0, IDXPAD // 16, init_body, jnp.int32(0))

        def comp_body(j, carry):
            off, tie = carry
            s = sv[pl.ds(j * 16, 16)]
            key = jnp.where(s < 0, s ^ jnp.int32(0x7FFFFFFF), s)
            gt = key > thr
            eq = key == thr
            eqi = jnp.where(eq, one, zero)
            exc_eq = plsc.cumsum(eqi) - eqi
            take_eq = jnp.logical_and(eq, (exc_eq + tie) < bud)
            sel = jnp.logical_or(gt, take_eq)
            seli = jnp.where(sel, one, zero)
            pos = jnp.where(sel, off + (plsc.cumsum(seli) - seli),
                            jnp.int32(IDXPAD) + iota)
            plsc.store_scatter(idx_v, [pos], j * 16 + iota)
            return (off + jnp.sum(seli), tie + jnp.sum(eqi))

        lax.fori_loop(0, N // 16, comp_body, (jnp.int32(0), jnp.int32(0)))

        pltpu.sync_copy(idx_v.at[pl.ds(0, IDXPAD)], idx_hbm)


@functools.cache
def _sc_compact_build():
    return functools.partial(
        pl.kernel,
        mesh=plsc.VectorSubcoreMesh(core_axis_name="c", subcore_axis_name="s"),
        compiler_params=pltpu.CompilerParams(
            needs_layout_passes=False, use_tc_tiling_on_sc=False),
        out_type=[jax.ShapeDtypeStruct((IDXPAD,), jnp.int32)],
        scratch_types=[
            pltpu.VMEM((N + 32,), jnp.int32),
            pltpu.VMEM((IDXPAD + 16,), jnp.int32),
            pltpu.VMEM((IDXPAD,), jnp.int32),
        ],
    )(_sc_compact_body)


def _tc_gather_body(idx_ref, *refs):
    a_refs = refs[0:8]
    xg_refs = refs[8:16]
    b_ref, xp_ref = refs[16], refs[17]
    for r in range(8):
        b_ref[pl.ds(r, 1), :] = a_refs[r][0]
        xp_ref[pl.ds(r, 1), :] = xg_refs[r][0]


def _tc_gather(idx, a, xg):
    def a_map(r):
        return lambda i, idx_ref: (idx_ref[8 * i + r], 0, 0)

    grid_spec = pltpu.PrefetchScalarGridSpec(
        num_scalar_prefetch=1,
        grid=(NG,),
        in_specs=(
            [pl.BlockSpec((1, 1, N), a_map(r)) for r in range(8)]
            + [pl.BlockSpec((1, 1, F), a_map(r)) for r in range(8)]
        ),
        out_specs=[
            pl.BlockSpec((8, N), lambda i, idx_ref: (i, 0)),
            pl.BlockSpec((8, F), lambda i, idx_ref: (i, 0)),
        ],
    )
    ret = pl.pallas_call(
        _tc_gather_body,
        grid_spec=grid_spec,
        out_shape=[
            jax.ShapeDtypeStruct((K, N), jnp.float32),
            jax.ShapeDtypeStruct((K, F), jnp.float32),
        ],
    )
    a3 = a.reshape(N, 1, N)
    xg3 = xg.reshape(N, 1, F)
    ops = lax.optimization_barrier(tuple([a3] * 8 + [xg3] * 8))
    return ret(idx, *ops)


def _sc_colgather_body(b3_hbm, idx_hbm, ap3_hbm, idx_v, rowbuf, outbuf):
    cid = lax.axis_index("c")
    sid = lax.axis_index("s")
    w = sid * 2 + cid

    pltpu.sync_copy(idx_hbm, idx_v)
    gbase = (NG * w) // NW

    def grp_body(t, carry):
        g = gbase + t
        pltpu.sync_copy(b3_hbm.at[g], rowbuf)

        def col_body(j, c):
            civ = idx_v[pl.ds(j * 16, 16)]
            for r in range(8):
                rv = jnp.broadcast_to(jnp.int32(r), (16,))
                outbuf[r, pl.ds(j * 16, 16)] = plsc.load_gather(
                    rowbuf, [rv, civ])
            return c

        lax.fori_loop(0, K // 16, col_body, jnp.int32(0))
        iota = lax.iota(jnp.int32, 16)
        tpos = jnp.minimum((K // 16) * 16 + iota, jnp.int32(K - 1))
        tciv = plsc.load_gather(idx_v, [tpos])
        for r in range(8):
            rv = jnp.broadcast_to(jnp.int32(r), (16,))
            tvals = plsc.load_gather(rowbuf, [rv, tciv])
            plsc.store_scatter(outbuf, [rv, tpos], tvals)
        pltpu.sync_copy(outbuf, ap3_hbm.at[g])
        return carry

    lax.fori_loop(0, (NG + NW - 1) // NW, grp_body, jnp.int32(0))


@functools.cache
def _sc_colgather_build():
    return functools.partial(
        pl.kernel,
        mesh=plsc.VectorSubcoreMesh(core_axis_name="c", subcore_axis_name="s"),
        compiler_params=pltpu.CompilerParams(needs_layout_passes=False),
        out_type=[jax.ShapeDtypeStruct((NG, 8, K), jnp.float32)],
        scratch_types=[
            pltpu.VMEM((IDXPAD,), jnp.int32),
            pltpu.VMEM((8, N), jnp.float32),
            pltpu.VMEM((8, K), jnp.float32),
        ],
    )(_sc_colgather_body)


def kernel(X, A, kernel):
    xg, y = _score_gate(X, kernel)
    meta = _threshold(y.reshape(10, 1000))
    y_bits = lax.bitcast_convert_type(y.reshape(N), jnp.int32)
    thr16 = jnp.full((16,), meta[0, 0], jnp.int32)
    bud16 = jnp.full((16,), meta[1, 0], jnp.int32)
    sm = jnp.concatenate([y_bits, thr16, bud16])
    idx = _one_out(_sc_compact_build()(sm))
    b, xp = _tc_gather(idx, A, xg)
    ap3 = _one_out(_sc_colgather_build()(b.reshape(NG, 8, N), idx))
    return (xp, ap3.reshape(K, K))
